# initial kernel scaffold (unmeasured)
import jax
import jax.numpy as jnp
from jax import lax
from jax.experimental import pallas as pl
from jax.experimental.pallas import tpu as pltpu

N_DEV = 4


def _allgather_partials(partial):
    m, n = partial.shape

    def body(p_ref, out_ref, send_sems, recv_sems, local_sem):
        my = lax.axis_index("i")

        barrier = pltpu.get_barrier_semaphore()
        for d in range(1, N_DEV):
            pl.semaphore_signal(
                barrier, inc=1,
                device_id=((my + d) % N_DEV,),
                device_id_type=pl.DeviceIdType.MESH,
            )
        pl.semaphore_wait(barrier, N_DEV - 1)

        cp = pltpu.make_async_copy(p_ref, out_ref.at[my], local_sem)
        cp.start()

        sends = []
        for d in range(1, N_DEV):
            peer = (my + d) % N_DEV
            rdma = pltpu.make_async_remote_copy(
                src_ref=p_ref,
                dst_ref=out_ref.at[my],
                send_sem=send_sems.at[d - 1],
                recv_sem=recv_sems.at[d - 1],
                device_id=(peer,),
                device_id_type=pl.DeviceIdType.MESH,
            )
            rdma.start()
            sends.append(rdma)

        cp.wait()
        for r in sends:
            r.wait_send()

        for d in range(1, N_DEV):
            src_pos = (my - d) % N_DEV
            recv = pltpu.make_async_remote_copy(
                src_ref=p_ref,
                dst_ref=out_ref.at[src_pos],
                send_sem=send_sems.at[0],
                recv_sem=recv_sems.at[d - 1],
                device_id=(src_pos,),
                device_id_type=pl.DeviceIdType.MESH,
            )
            recv.wait_recv()

    return pl.pallas_call(
        body,
        out_shape=jax.ShapeDtypeStruct((N_DEV, m, n), partial.dtype),
        in_specs=[pl.BlockSpec(memory_space=pltpu.ANY)],
        out_specs=pl.BlockSpec(memory_space=pltpu.ANY),
        scratch_shapes=[
            pltpu.SemaphoreType.DMA((N_DEV - 1,)),
            pltpu.SemaphoreType.DMA((N_DEV - 1,)),
            pltpu.SemaphoreType.DMA,
        ],
        compiler_params=pltpu.CompilerParams(collective_id=0),
    )(partial)


def kernel(x, w_mat):
    partial = jnp.dot(x, w_mat, preferred_element_type=jnp.float32)
    gathered = _allgather_partials(partial)
    y = jnp.maximum(gathered.sum(axis=0), 0.0)
    scale = jnp.max(y) / 448.0
    q = (y / scale).astype(jnp.float8_e4m3fn)
    return q.astype(jnp.float32) * scale


# baseline (device time: 899743 ns/iter reference)
import jax
import jax.numpy as jnp
from jax import lax
from jax.experimental import pallas as pl
from jax.experimental.pallas import tpu as pltpu

N_DEV = 4


def _push_allgather(arr, collective_id, scatter):
    chunk_shape = arr.shape[1:] if scatter else arr.shape

    def body(a_ref, out_ref, send_sems, recv_sems, local_sem):
        my = lax.axis_index("i")

        barrier = pltpu.get_barrier_semaphore()
        for d in range(1, N_DEV):
            pl.semaphore_signal(
                barrier, inc=1,
                device_id=((my + d) % N_DEV,),
                device_id_type=pl.DeviceIdType.MESH,
            )
        pl.semaphore_wait(barrier, N_DEV - 1)

        src_local = a_ref.at[my] if scatter else a_ref
        cp = pltpu.make_async_copy(src_local, out_ref.at[my], local_sem)
        cp.start()

        sends = []
        for d in range(1, N_DEV):
            peer = (my + d) % N_DEV
            rdma = pltpu.make_async_remote_copy(
                src_ref=a_ref.at[peer] if scatter else a_ref,
                dst_ref=out_ref.at[my],
                send_sem=send_sems.at[d - 1],
                recv_sem=recv_sems.at[d - 1],
                device_id=(peer,),
                device_id_type=pl.DeviceIdType.MESH,
            )
            rdma.start()
            sends.append(rdma)

        cp.wait()
        for r in sends:
            r.wait_send()

        for d in range(1, N_DEV):
            src_pos = (my - d) % N_DEV
            recv = pltpu.make_async_remote_copy(
                src_ref=a_ref.at[src_pos] if scatter else a_ref,
                dst_ref=out_ref.at[src_pos],
                send_sem=send_sems.at[0],
                recv_sem=recv_sems.at[d - 1],
                device_id=(src_pos,),
                device_id_type=pl.DeviceIdType.MESH,
            )
            recv.wait_recv()

    return pl.pallas_call(
        body,
        out_shape=jax.ShapeDtypeStruct((N_DEV, *chunk_shape), arr.dtype),
        in_specs=[pl.BlockSpec(memory_space=pl.ANY)],
        out_specs=pl.BlockSpec(memory_space=pl.ANY),
        scratch_shapes=[
            pltpu.SemaphoreType.DMA((N_DEV - 1,)),
            pltpu.SemaphoreType.DMA((N_DEV - 1,)),
            pltpu.SemaphoreType.DMA,
        ],
        compiler_params=pltpu.CompilerParams(collective_id=collective_id),
    )(arr)


def kernel(x, w_mat):
    m = x.shape[0]
    n = w_mat.shape[1]
    m_chunk = m // N_DEV

    partial = jnp.dot(x, w_mat, preferred_element_type=jnp.float32)

    p_bf16 = partial.reshape(N_DEV, m_chunk, n).astype(jnp.bfloat16)
    recv = _push_allgather(p_bf16, collective_id=0, scatter=True)
    my_chunk = jnp.maximum(recv.astype(jnp.float32).sum(axis=0), 0.0)

    amax_tile = jnp.full((8, 128), jnp.max(my_chunk), jnp.float32)
    amaxes = _push_allgather(amax_tile, collective_id=1, scatter=False)
    scale = jnp.max(amaxes) / 448.0

    q = (my_chunk / scale).astype(jnp.float8_e4m3fn)
    q_all = _push_allgather(q, collective_id=2, scatter=False)
    return q_all.reshape(m, n).astype(jnp.float32) * scale


# device time: 762867 ns/iter; 1.1794x vs baseline; 1.1794x over previous
import jax
import jax.numpy as jnp
from jax import lax
from jax.experimental import pallas as pl
from jax.experimental.pallas import tpu as pltpu

N_DEV = 4


def _push_allgather(arr, collective_id, scatter):
    chunk_shape = arr.shape[1:] if scatter else arr.shape

    def body(a_ref, out_ref, send_sems, recv_sems, local_sem):
        my = lax.axis_index("i")

        barrier = pltpu.get_barrier_semaphore()
        for d in range(1, N_DEV):
            pl.semaphore_signal(
                barrier, inc=1,
                device_id=((my + d) % N_DEV,),
                device_id_type=pl.DeviceIdType.MESH,
            )
        pl.semaphore_wait(barrier, N_DEV - 1)

        src_local = a_ref.at[my] if scatter else a_ref
        cp = pltpu.make_async_copy(src_local, out_ref.at[my], local_sem)
        cp.start()

        sends = []
        for d in range(1, N_DEV):
            peer = (my + d) % N_DEV
            rdma = pltpu.make_async_remote_copy(
                src_ref=a_ref.at[peer] if scatter else a_ref,
                dst_ref=out_ref.at[my],
                send_sem=send_sems.at[d - 1],
                recv_sem=recv_sems.at[d - 1],
                device_id=(peer,),
                device_id_type=pl.DeviceIdType.MESH,
            )
            rdma.start()
            sends.append(rdma)

        cp.wait()
        for r in sends:
            r.wait_send()

        for d in range(1, N_DEV):
            src_pos = (my - d) % N_DEV
            recv = pltpu.make_async_remote_copy(
                src_ref=a_ref.at[src_pos] if scatter else a_ref,
                dst_ref=out_ref.at[src_pos],
                send_sem=send_sems.at[0],
                recv_sem=recv_sems.at[d - 1],
                device_id=(src_pos,),
                device_id_type=pl.DeviceIdType.MESH,
            )
            recv.wait_recv()

    return pl.pallas_call(
        body,
        out_shape=jax.ShapeDtypeStruct((N_DEV, *chunk_shape), arr.dtype),
        in_specs=[pl.BlockSpec(memory_space=pl.ANY)],
        out_specs=pl.BlockSpec(memory_space=pl.ANY),
        scratch_shapes=[
            pltpu.SemaphoreType.DMA((N_DEV - 1,)),
            pltpu.SemaphoreType.DMA((N_DEV - 1,)),
            pltpu.SemaphoreType.DMA,
        ],
        compiler_params=pltpu.CompilerParams(collective_id=collective_id),
    )(arr)


def _ring_hop(send_cw, send_ccw, collective_id):

    def body(cw_ref, ccw_ref, rcw_ref, rccw_ref, send_sems, recv_sems):
        my = lax.axis_index("i")
        right = (my + 1) % N_DEV
        left = (my - 1) % N_DEV

        barrier = pltpu.get_barrier_semaphore()
        for nbr in (left, right):
            pl.semaphore_signal(
                barrier, inc=1,
                device_id=(nbr,), device_id_type=pl.DeviceIdType.MESH,
            )
        pl.semaphore_wait(barrier, 2)

        cw = pltpu.make_async_remote_copy(
            src_ref=cw_ref, dst_ref=rcw_ref,
            send_sem=send_sems.at[0], recv_sem=recv_sems.at[0],
            device_id=(right,), device_id_type=pl.DeviceIdType.MESH,
        )
        ccw = pltpu.make_async_remote_copy(
            src_ref=ccw_ref, dst_ref=rccw_ref,
            send_sem=send_sems.at[1], recv_sem=recv_sems.at[1],
            device_id=(left,), device_id_type=pl.DeviceIdType.MESH,
        )
        cw.start()
        ccw.start()
        cw.wait_send()
        ccw.wait_send()
        cw.wait_recv()
        ccw.wait_recv()

    shape = jax.ShapeDtypeStruct(send_cw.shape, send_cw.dtype)
    return pl.pallas_call(
        body,
        out_shape=(shape, shape),
        in_specs=[
            pl.BlockSpec(memory_space=pl.ANY),
            pl.BlockSpec(memory_space=pl.ANY),
        ],
        out_specs=(
            pl.BlockSpec(memory_space=pl.ANY),
            pl.BlockSpec(memory_space=pl.ANY),
        ),
        scratch_shapes=[
            pltpu.SemaphoreType.DMA((2,)),
            pltpu.SemaphoreType.DMA((2,)),
        ],
        compiler_params=pltpu.CompilerParams(collective_id=collective_id),
    )(send_cw, send_ccw)


def kernel(x, w_mat):
    m = x.shape[0]
    n = w_mat.shape[1]
    m_chunk = m // N_DEV
    half = n // 2

    my = lax.axis_index("i")
    partial = jnp.dot(x, w_mat, preferred_element_type=jnp.float32)

    p16 = partial.reshape(N_DEV, m_chunk, n).astype(jnp.bfloat16)

    def chunk16(c):
        return lax.dynamic_index_in_dim(p16, c % N_DEV, axis=0, keepdims=False)

    acc_cw = chunk16(my - 1)[:, :half]
    acc_ccw = chunk16(my + 1)[:, half:]
    for h in range(N_DEV - 1):
        r_cw, r_ccw = _ring_hop(acc_cw, acc_ccw, collective_id=h)
        src = chunk16(my - 2 - h)
        acc_cw = r_cw + src[:, :half]
        src = chunk16(my + 2 + h)
        acc_ccw = r_ccw + src[:, half:]
    my_chunk = jnp.concatenate([acc_cw, acc_ccw], axis=1).astype(jnp.float32)
    my_chunk = jnp.maximum(my_chunk, 0.0)

    amax_tile = jnp.full((8, 128), jnp.max(my_chunk), jnp.float32)
    amaxes = _push_allgather(amax_tile, collective_id=3, scatter=False)
    scale = jnp.max(amaxes) / 448.0

    q = (my_chunk / scale).astype(jnp.float8_e4m3fn)
    q_all = _push_allgather(q, collective_id=4, scatter=False)
    return q_all.reshape(m, n).astype(jnp.float32) * scale
